# Initial kernel scaffold; baseline (speedup 1.0000x reference)
import jax
import jax.numpy as jnp
from jax.experimental import pallas as pl


def kernel(embedding, fc_w, fc_b, w_ih_0, w_hh_0, b_0, w_ih_1, w_hh_1, b_1, x_tokens, h0, c0):
    raise NotImplementedError("write your pallas kernel here")


# speedup vs baseline: 1.4852x; 1.4852x over previous
import functools

import jax
import jax.numpy as jnp
from jax import lax
from jax.experimental import pallas as pl
from jax.experimental.pallas import tpu as pltpu

_H = 32
_LAYERS = 2


def _round_up(x, m):
    return ((x + m - 1) // m) * m


def _lstm_cell(tg, c_prev, H):
    i_g = 0.5 * (tg[:, 0 * H:1 * H] + 1.0)
    f_g = 0.5 * (tg[:, 1 * H:2 * H] + 1.0)
    g_g = tg[:, 2 * H:3 * H]
    o_g = 0.5 * (tg[:, 3 * H:4 * H] + 1.0)
    c_new = f_g * c_prev + i_g * g_g
    h_new = o_g * jnp.tanh(c_new)
    return h_new, c_new


def _rnn_kernel(tok_ref, wx0_ref, b0_ref, whh0_ref, wih1_ref, whh1_ref,
                b1_ref, wfc_ref, bfc_ref, h0_ref, c0_ref,
                logits_ref, hN_ref, cN_ref,
                xg_scr, seq_scr, h0_scr, c0_scr, h1_scr, c1_scr,
                *, Tc, H, V):
    t = pl.program_id(1)
    Bpb = h0_scr.shape[0]
    H4 = 4 * H
    rows = Tc * Bpb

    @pl.when(t == 0)
    def _():
        h0_scr[...] = h0_ref[0]
        c0_scr[...] = c0_ref[0]
        h1_scr[...] = h0_ref[1]
        c1_scr[...] = c0_ref[1]

    tok = tok_ref[0]
    oh = (tok == lax.broadcasted_iota(jnp.int32, (rows, V), 1)).astype(jnp.bfloat16)
    xg_scr[...] = (jnp.dot(oh, wx0_ref[...], preferred_element_type=jnp.float32)
                   + b0_ref[...])

    h0c = h0_scr[...]
    g0 = jnp.dot(h0c, whh0_ref[...], preferred_element_type=jnp.float32)
    tg0 = jnp.tanh(g0 + xg_scr[pl.ds(0, Bpb), :])
    h0n, c0n = _lstm_cell(tg0, c0_scr[...], H)
    h0_scr[...] = h0n
    c0_scr[...] = c0n

    def body(k, carry):
        r = pl.multiple_of(k * Bpb, Bpb)
        rp = pl.multiple_of((k - 1) * Bpb, Bpb)
        h0c = h0_scr[...]
        h1c = h1_scr[...]
        g1 = (jnp.dot(h0c, wih1_ref[...], preferred_element_type=jnp.float32)
              + jnp.dot(h1c, whh1_ref[...], preferred_element_type=jnp.float32))
        g0 = jnp.dot(h0c, whh0_ref[...], preferred_element_type=jnp.float32)
        tg0 = jnp.tanh(g0 + xg_scr[pl.ds(r, Bpb), :])
        tg1 = jnp.tanh(g1 + b1_ref[...])
        h0n, c0n = _lstm_cell(tg0, c0_scr[...], H)
        h1n, c1n = _lstm_cell(tg1, c1_scr[...], H)
        h0_scr[...] = h0n
        c0_scr[...] = c0n
        h1_scr[...] = h1n
        c1_scr[...] = c1n
        seq_scr[pl.ds(rp, Bpb), :] = h1n
        return carry

    unroll = True if Tc <= 32 else 8
    lax.fori_loop(1, Tc, body, 0, unroll=unroll)

    h0c = h0_scr[...]
    h1c = h1_scr[...]
    g1 = (jnp.dot(h0c, wih1_ref[...], preferred_element_type=jnp.float32)
          + jnp.dot(h1c, whh1_ref[...], preferred_element_type=jnp.float32))
    tg1 = jnp.tanh(g1 + b1_ref[...])
    h1n, c1n = _lstm_cell(tg1, c1_scr[...], H)
    h1_scr[...] = h1n
    c1_scr[...] = c1n
    seq_scr[pl.ds((Tc - 1) * Bpb, Bpb), :] = h1n

    lg = (jnp.dot(seq_scr[...].astype(jnp.bfloat16), wfc_ref[...],
                  preferred_element_type=jnp.float32) + bfc_ref[...])
    logits_ref[0] = lg

    hN_ref[0] = h0_scr[...]
    hN_ref[1] = h1_scr[...]
    cN_ref[0] = c0_scr[...]
    cN_ref[1] = c1_scr[...]


def _rnn_call(tok_flat, wx0, b0, whh0, wih1, whh1, b1, wfc, bfc, h0, c0,
              *, Tc, Bpb, NB, H, V):
    TBpb = tok_flat.shape[1]
    T = TBpb // Bpb
    n_chunks = T // Tc
    rows = Tc * Bpb
    H4 = 4 * H
    L = h0.shape[0]
    Bp = h0.shape[1]

    def const(shape):
        return pl.BlockSpec(shape, lambda b, t, _n=len(shape): (0,) * _n)

    kernel_fn = functools.partial(_rnn_kernel, Tc=Tc, H=H, V=V)

    out_shapes = (
        jax.ShapeDtypeStruct((NB, TBpb, V), jnp.float32),
        jax.ShapeDtypeStruct((L, Bp, H), jnp.float32),
        jax.ShapeDtypeStruct((L, Bp, H), jnp.float32),
    )

    return pl.pallas_call(
        kernel_fn,
        out_shape=out_shapes,
        grid=(NB, n_chunks),
        in_specs=[
            pl.BlockSpec((1, rows, 1), lambda b, t: (b, t, 0)),
            const((V, H4)),
            const((1, H4)),
            const((H, H4)),
            const((H, H4)),
            const((H, H4)),
            const((1, H4)),
            const((H, V)),
            const((1, V)),
            pl.BlockSpec((L, Bpb, H), lambda b, t: (0, b, 0)),
            pl.BlockSpec((L, Bpb, H), lambda b, t: (0, b, 0)),
        ],
        out_specs=[
            pl.BlockSpec((1, rows, V), lambda b, t: (b, t, 0)),
            pl.BlockSpec((L, Bpb, H), lambda b, t: (0, b, 0)),
            pl.BlockSpec((L, Bpb, H), lambda b, t: (0, b, 0)),
        ],
        scratch_shapes=[
            pltpu.VMEM((rows, H4), jnp.float32),
            pltpu.VMEM((rows, H), jnp.float32),
            pltpu.VMEM((Bpb, H), jnp.float32),
            pltpu.VMEM((Bpb, H), jnp.float32),
            pltpu.VMEM((Bpb, H), jnp.float32),
            pltpu.VMEM((Bpb, H), jnp.float32),
        ],
        compiler_params=pltpu.CompilerParams(
            dimension_semantics=("parallel", "arbitrary"),
            vmem_limit_bytes=64 << 20),
    )(tok_flat, wx0, b0, whh0, wih1, whh1, b1, wfc, bfc, h0, c0)


def kernel(embedding, fc_w, fc_b, w_ih_0, w_hh_0, b_0,
           w_ih_1, w_hh_1, b_1, x_tokens, h0, c0):
    B, T = x_tokens.shape
    H = _H
    V = fc_w.shape[1]

    Bp = _round_up(B, 8)
    NB = 2 if (Bp % 16 == 0 and Bp >= 16) else 1
    Bpb = Bp // NB
    Tc = 32
    while T % Tc:
        Tc //= 2

    scale = jnp.concatenate([
        jnp.full((2 * H,), 0.5, jnp.float32),
        jnp.ones((H,), jnp.float32),
        jnp.full((H,), 0.5, jnp.float32),
    ])[None, :]

    wx0 = jnp.dot(embedding, w_ih_0 * scale).astype(jnp.bfloat16)
    b0 = (b_0 * scale).astype(jnp.float32)
    whh0 = (w_hh_0 * scale).astype(jnp.float32)
    wih1 = (w_ih_1 * scale).astype(jnp.float32)
    whh1 = (w_hh_1 * scale).astype(jnp.float32)
    b1 = (b_1 * scale).astype(jnp.float32)
    wfc = fc_w.astype(jnp.bfloat16)
    bfc = fc_b.reshape(1, V).astype(jnp.float32)

    tok_t = x_tokens.T
    if Bp != B:
        tok_t = jnp.pad(tok_t, ((0, 0), (0, Bp - B)))
    tok_flat = (tok_t.reshape(T, NB, Bpb).swapaxes(0, 1)
                .reshape(NB, T * Bpb, 1))
    h0_p = h0.astype(jnp.float32)
    c0_p = c0.astype(jnp.float32)
    if Bp != B:
        h0_p = jnp.pad(h0_p, ((0, 0), (0, Bp - B), (0, 0)))
        c0_p = jnp.pad(c0_p, ((0, 0), (0, Bp - B), (0, 0)))

    logits3, hN, cN = _rnn_call(
        tok_flat, wx0, b0, whh0, wih1, whh1, b1, wfc, bfc, h0_p, c0_p,
        Tc=Tc, Bpb=Bpb, NB=NB, H=H, V=V)

    logits = (logits3.reshape(NB, T, Bpb, V).transpose(0, 2, 1, 3)
              .reshape(Bp, T, V)[:B].reshape(B * T, V))
    return logits, (hN[:, :B, :], cN[:, :B, :])


# speedup vs baseline: 4.1638x; 2.8034x over previous
import functools

import jax
import jax.numpy as jnp
from jax import lax
from jax.experimental import pallas as pl
from jax.experimental.pallas import tpu as pltpu

_H = 32
_LAYERS = 2


def _round_up(x, m):
    return ((x + m - 1) // m) * m


def _lstm_cell_t(tg, c_prev, H):
    ti = tg[0 * H:1 * H]
    tf = tg[1 * H:2 * H]
    gg = tg[2 * H:3 * H]
    to = tg[3 * H:4 * H]
    c_new = 0.5 * ((tf + 1.0) * c_prev + (ti + 1.0) * gg)
    h_new = (0.5 * (to + 1.0)) * jnp.tanh(c_new)
    return h_new, c_new


def _rnn_kernel(tok_ref, wx0_ref, whh0_ref, wih1_ref, whh1a_ref,
                wfc_ref, bfc_ref, h0_ref, c0_ref,
                logits_ref, hN_ref, cN_ref,
                xg_scr, seq_scr, h0_scr, c0_scr, hc1_scr, c1_scr,
                *, Tc, H, V):
    t = pl.program_id(1)
    Bpb = h0_scr.shape[1]
    H4 = 4 * H
    rows = Tc * Bpb

    @pl.when(t == 0)
    def _():
        h0_scr[...] = h0_ref[0]
        c0_scr[...] = c0_ref[0]
        hc1_scr[pl.ds(0, H), :] = h0_ref[1]
        c1_scr[...] = c0_ref[1]
        pad = hc1_scr.shape[0] - H
        hc1_scr[pl.ds(H, pad), :] = (
            lax.broadcasted_iota(jnp.int32, (pad, Bpb), 0) == 0
        ).astype(jnp.float32)

    tok = tok_ref[0, 0]
    oh = (lax.broadcasted_iota(jnp.int32, (V, rows), 0) == tok).astype(jnp.bfloat16)
    xg_scr[...] = jnp.dot(wx0_ref[...], oh, preferred_element_type=jnp.float32)

    h0c = h0_scr[...]
    g0 = jnp.dot(whh0_ref[...], h0c, preferred_element_type=jnp.float32)
    tg0 = jnp.tanh(g0 + xg_scr[:, pl.ds(0, Bpb)])
    h0n, c0n = _lstm_cell_t(tg0, c0_scr[...], H)
    h0_scr[...] = h0n
    c0_scr[...] = c0n

    def body(k, carry):
        r = pl.multiple_of(k * Bpb, Bpb)
        rp = pl.multiple_of((k - 1) * Bpb, Bpb)
        h0c = h0_scr[...]
        g1 = (jnp.dot(wih1_ref[...], h0c, preferred_element_type=jnp.float32)
              + jnp.dot(whh1a_ref[...], hc1_scr[...],
                        preferred_element_type=jnp.float32))
        g0 = jnp.dot(whh0_ref[...], h0c, preferred_element_type=jnp.float32)
        tg0 = jnp.tanh(g0 + xg_scr[:, pl.ds(r, Bpb)])
        tg1 = jnp.tanh(g1)
        h0n, c0n = _lstm_cell_t(tg0, c0_scr[...], H)
        h1n, c1n = _lstm_cell_t(tg1, c1_scr[...], H)
        h0_scr[...] = h0n
        c0_scr[...] = c0n
        hc1_scr[pl.ds(0, H), :] = h1n
        c1_scr[...] = c1n
        seq_scr[:, pl.ds(rp, Bpb)] = h1n
        return carry

    unroll = True if Tc <= 32 else 8
    lax.fori_loop(1, Tc, body, 0, unroll=unroll)

    h0c = h0_scr[...]
    g1 = (jnp.dot(wih1_ref[...], h0c, preferred_element_type=jnp.float32)
          + jnp.dot(whh1a_ref[...], hc1_scr[...],
                    preferred_element_type=jnp.float32))
    tg1 = jnp.tanh(g1)
    h1n, c1n = _lstm_cell_t(tg1, c1_scr[...], H)
    hc1_scr[pl.ds(0, H), :] = h1n
    c1_scr[...] = c1n
    seq_scr[:, pl.ds((Tc - 1) * Bpb, Bpb)] = h1n

    lg = lax.dot_general(seq_scr[...].astype(jnp.bfloat16), wfc_ref[...],
                         (((0,), (0,)), ((), ())),
                         preferred_element_type=jnp.float32) + bfc_ref[...]
    logits_ref[0] = lg

    hN_ref[0] = h0_scr[...]
    hN_ref[1] = hc1_scr[pl.ds(0, H), :]
    cN_ref[0] = c0_scr[...]
    cN_ref[1] = c1_scr[...]


def _rnn_call(tok4, wx0, whh0, wih1, whh1a, wfc, bfc, h0, c0,
              *, Tc, Bpb, NB, H, V):
    n_chunks = tok4.shape[1]
    rows = Tc * Bpb
    T = n_chunks * Tc
    H4 = 4 * H
    L = h0.shape[0]
    Bp = h0.shape[2]
    Ha = whh1a.shape[1]

    def const(shape):
        return pl.BlockSpec(shape, lambda b, t, _n=len(shape): (0,) * _n)

    kernel_fn = functools.partial(_rnn_kernel, Tc=Tc, H=H, V=V)

    out_shapes = (
        jax.ShapeDtypeStruct((NB, T * Bpb, V), jnp.float32),
        jax.ShapeDtypeStruct((L, H, Bp), jnp.float32),
        jax.ShapeDtypeStruct((L, H, Bp), jnp.float32),
    )

    return pl.pallas_call(
        kernel_fn,
        out_shape=out_shapes,
        grid=(NB, n_chunks),
        in_specs=[
            pl.BlockSpec((1, 1, 1, rows), lambda b, t: (b, t, 0, 0)),
            const((H4, V)),
            const((H4, H)),
            const((H4, H)),
            const((H4, Ha)),
            const((H, V)),
            const((1, V)),
            pl.BlockSpec((L, H, Bpb), lambda b, t: (0, 0, b)),
            pl.BlockSpec((L, H, Bpb), lambda b, t: (0, 0, b)),
        ],
        out_specs=[
            pl.BlockSpec((1, rows, V), lambda b, t: (b, t, 0)),
            pl.BlockSpec((L, H, Bpb), lambda b, t: (0, 0, b)),
            pl.BlockSpec((L, H, Bpb), lambda b, t: (0, 0, b)),
        ],
        scratch_shapes=[
            pltpu.VMEM((H4, rows), jnp.float32),
            pltpu.VMEM((H, rows), jnp.float32),
            pltpu.VMEM((H, Bpb), jnp.float32),
            pltpu.VMEM((H, Bpb), jnp.float32),
            pltpu.VMEM((Ha, Bpb), jnp.float32),
            pltpu.VMEM((H, Bpb), jnp.float32),
        ],
        compiler_params=pltpu.CompilerParams(
            dimension_semantics=("parallel", "arbitrary"),
            vmem_limit_bytes=64 << 20),
    )(tok4, wx0, whh0, wih1, whh1a, wfc, bfc, h0, c0)


def kernel(embedding, fc_w, fc_b, w_ih_0, w_hh_0, b_0,
           w_ih_1, w_hh_1, b_1, x_tokens, h0, c0):
    B, T = x_tokens.shape
    H = _H
    V = fc_w.shape[1]
    H4 = 4 * H

    Bp = _round_up(B, 8)
    NB = 2 if (Bp % 16 == 0 and Bp >= 16) else 1
    Bpb = Bp // NB
    Tc = 32
    while T % Tc:
        Tc //= 2
    n_chunks = T // Tc
    rows = Tc * Bpb

    scale = jnp.concatenate([
        jnp.full((2 * H,), 0.5, jnp.float32),
        jnp.ones((H,), jnp.float32),
        jnp.full((H,), 0.5, jnp.float32),
    ])[None, :]

    wx0 = (jnp.dot(embedding, w_ih_0 * scale) + b_0 * scale).T.astype(jnp.bfloat16)
    whh0 = (w_hh_0 * scale).T.astype(jnp.float32)
    wih1 = (w_ih_1 * scale).T.astype(jnp.float32)
    whh1a = jnp.concatenate([
        (w_hh_1 * scale).T,
        (b_1 * scale).reshape(H4, 1),
        jnp.zeros((H4, 7), jnp.float32),
    ], axis=1).astype(jnp.float32)
    wfc = fc_w.astype(jnp.bfloat16)
    bfc = fc_b.reshape(1, V).astype(jnp.float32)

    tok_t = x_tokens.T
    if Bp != B:
        tok_t = jnp.pad(tok_t, ((0, 0), (0, Bp - B)))
    tok4 = (tok_t.reshape(T, NB, Bpb).swapaxes(0, 1)
            .reshape(NB, n_chunks, 1, rows))
    h0_p = h0.astype(jnp.float32)
    c0_p = c0.astype(jnp.float32)
    if Bp != B:
        h0_p = jnp.pad(h0_p, ((0, 0), (0, Bp - B), (0, 0)))
        c0_p = jnp.pad(c0_p, ((0, 0), (0, Bp - B), (0, 0)))
    h0_t = h0_p.transpose(0, 2, 1)
    c0_t = c0_p.transpose(0, 2, 1)

    logits3, hN_t, cN_t = _rnn_call(
        tok4, wx0, whh0, wih1, whh1a, wfc, bfc, h0_t, c0_t,
        Tc=Tc, Bpb=Bpb, NB=NB, H=H, V=V)

    logits = (logits3.reshape(NB, T, Bpb, V).transpose(0, 2, 1, 3)
              .reshape(Bp, T, V)[:B].reshape(B * T, V))
    hN = hN_t.transpose(0, 2, 1)[:, :B, :]
    cN = cN_t.transpose(0, 2, 1)[:, :B, :]
    return logits, (hN, cN)
